# Initial kernel scaffold; baseline (speedup 1.0000x reference)
import jax
import jax.numpy as jnp
from jax.experimental import pallas as pl


def kernel(preds, labels, images):
    raise NotImplementedError("write your pallas kernel here")


# speedup vs baseline: 127.9412x; 127.9412x over previous
import numpy as np
import jax
import jax.numpy as jnp
from jax.experimental import pallas as pl
from jax.experimental.pallas import tpu as pltpu

_H = 512
_W = 512
_P = _H * _W
_NB = 256
_SCALE = np.float32(256.0 / 255.0)


def _shift0(x, s):
    if s == 0:
        return x
    z = jnp.zeros((abs(s),) + x.shape[1:], x.dtype)
    if s > 0:
        return jnp.concatenate([x[s:], z], axis=0)
    return jnp.concatenate([z, x[:s]], axis=0)


def _shift1(x, s):
    if s == 0:
        return x
    z = jnp.zeros(x.shape[:1] + (abs(s),), x.dtype)
    if s > 0:
        return jnp.concatenate([x[:, s:], z], axis=1)
    return jnp.concatenate([z, x[:, :s]], axis=1)


def _cumsum1(v):
    acc = v
    k = 1
    while k < _NB:
        acc = acc + _shift1(acc, -k)
        k *= 2
    return acc


def _otsu(hist_row):
    S = jnp.sum(hist_row)
    ar_row = jax.lax.broadcasted_iota(jnp.int32, (1, _NB), 1).astype(jnp.float32)
    ch_row = _cumsum1(hist_row) / S
    cm_row = _cumsum1(hist_row * ar_row) / S
    tm = jax.lax.slice(cm_row, (0, _NB - 1), (1, _NB))

    chj = jnp.broadcast_to(ch_row, (_NB, _NB))
    chi = chj.T
    cmj = jnp.broadcast_to(cm_row, (_NB, _NB))
    cmi = cmj.T

    s = 1e-08
    w0 = chi
    w1 = chj - w0
    w2 = 1.0 - chj
    m0 = cmi / (w0 + s)
    m1 = (cmj - cmi) / (w1 + s)
    m2 = (tm - cmj) / (w2 + s)
    valid = (w0 > 0) & (w1 > 0) & (w2 > 0)
    bv = w0 * (m0 - tm) ** 2 + w1 * (m1 - tm) ** 2 + w2 * (m2 - tm) ** 2

    ii = jax.lax.broadcasted_iota(jnp.int32, (_NB, _NB), 0)
    jj = jax.lax.broadcasted_iota(jnp.int32, (_NB, _NB), 1)
    inrange = (ii < _NB - 2) & (jj < _NB - 2)
    bvm = jnp.where(valid & inrange, bv, jnp.where(inrange, 0.0, -1.0))
    maxbv = jnp.max(bvm)
    r = ii * 65536 + jj
    rmin = jnp.min(jnp.where(bvm == maxbv, r, jnp.int32(2 ** 30)))
    t1 = 1 + (rmin >> 16)
    t2 = 1 + (rmin & 65535)
    return t1, t2


def _body(preds_ref, labels_ref, images_ref, ssd_ref, sdl_ref):
    lab = labels_ref[0, 0]
    img = images_ref[0, 0]
    prd = preds_ref[0, 0]

    mh = lab
    for s in (-2, -1, 1, 2):
        mh = jnp.maximum(mh, _shift1(lab, s))
    mv = mh
    for s in (-2, -1, 1, 2):
        mv = jnp.maximum(mv, _shift0(mh, s))
    dl = (mv > 0).astype(jnp.float32)

    x = img * dl
    v = x * 255.0
    h_idx = jnp.clip((v * _SCALE).astype(jnp.int32), 0, 255)
    k = jnp.clip(v.astype(jnp.int32), 0, 255)

    hif = h_idx.reshape(1, _P)
    dlf = dl.reshape(1, _P)
    c16 = jax.lax.broadcasted_iota(jnp.int32, (16, _P), 0)
    a = jnp.where((hif >> 4) == c16, dlf, 0.0).astype(jnp.bfloat16)
    b = ((hif & 15) == c16).astype(jnp.bfloat16)
    hist16 = jax.lax.dot_general(
        a, b, (((1,), (1,)), ((), ())), preferred_element_type=jnp.float32)
    hist_row = jnp.concatenate(
        [jax.lax.slice(hist16, (i, 0), (i + 1, 16)) for i in range(16)], axis=1)

    t1, t2 = _otsu(hist_row)

    qa = (k >= t1).astype(jnp.float32)
    qb = (k >= t2).astype(jnp.float32)
    q = (qa + qb) * 0.5
    d = q - prd * dl
    ssd = jnp.sum(d * d)
    sdl = jnp.sum(hist_row)

    ssd_ref[0, 0, :] = jnp.full((128,), ssd, jnp.float32)
    sdl_ref[0, 0, :] = jnp.full((128,), sdl, jnp.float32)


def kernel(preds, labels, images):
    n = preds.shape[0]
    ssd, sdl = pl.pallas_call(
        _body,
        grid=(n,),
        in_specs=[
            pl.BlockSpec((1, 1, _H, _W), lambda i: (i, 0, 0, 0)),
            pl.BlockSpec((1, 1, _H, _W), lambda i: (i, 0, 0, 0)),
            pl.BlockSpec((1, 1, _H, _W), lambda i: (i, 0, 0, 0)),
        ],
        out_specs=[
            pl.BlockSpec((1, 1, 128), lambda i: (i, 0, 0)),
            pl.BlockSpec((1, 1, 128), lambda i: (i, 0, 0)),
        ],
        out_shape=[
            jax.ShapeDtypeStruct((n, 1, 128), jnp.float32),
            jax.ShapeDtypeStruct((n, 1, 128), jnp.float32),
        ],
    )(preds, labels, images)
    ssd = ssd[:, 0, 0]
    sdl = sdl[:, 0, 0] + 1e-08
    valid = sdl > 1e-08
    nl = ssd / sdl
    denom = jnp.maximum(jnp.sum(valid.astype(jnp.float32)), 1.0)
    return jnp.sum(jnp.where(valid, nl, 0.0)) / denom
